# Initial kernel scaffold; baseline (speedup 1.0000x reference)
import jax
import jax.numpy as jnp
from jax.experimental import pallas as pl


def kernel(feat, edge_index, etypes, weight):
    raise NotImplementedError("write your pallas kernel here")


# speedup vs baseline: 13.8507x; 13.8507x over previous
import functools

import jax
import jax.numpy as jnp
from jax import lax
from jax.experimental import pallas as pl
from jax.experimental.pallas import tpu as pltpu
from jax.experimental.pallas import tpu_sc as plsc

N = 10000
E = 320000
F = 128
R = 16

NC = 2
NS = 16
NW = NC * NS
EDGES_PER_W = E // NW
CHUNK = 80
NCHUNK = EDGES_PER_W // CHUNK
ROWCHUNKS = N // CHUNK


def _relmm_body(feat_ref, w_ref, out_ref):
    out_ref[0] = jnp.dot(feat_ref[...], w_ref[0],
                         preferred_element_type=jnp.float32)


_BN = 1000


@jax.jit
def _relmm(feat, weight):
    return pl.pallas_call(
        _relmm_body,
        grid=(R, N // _BN),
        in_specs=[
            pl.BlockSpec((_BN, F), lambda r, i: (i, 0)),
            pl.BlockSpec((1, F, F), lambda r, i: (r, 0, 0)),
        ],
        out_specs=pl.BlockSpec((1, _BN, F), lambda r, i: (r, i, 0)),
        out_shape=jax.ShapeDtypeStruct((R, N, F), jnp.float32),
        compiler_params=pltpu.CompilerParams(
            dimension_semantics=("parallel", "parallel")),
    )(feat, weight)


_sc_mesh = plsc.VectorSubcoreMesh(
    core_axis_name="c", subcore_axis_name="s", num_cores=NC, num_subcores=NS)


@functools.partial(
    pl.kernel,
    out_type=jax.ShapeDtypeStruct((NC, N, F), jnp.float32),
    mesh=_sc_mesh,
    scratch_types=[
        pltpu.VMEM((CHUNK,), jnp.int32),
        pltpu.VMEM((CHUNK,), jnp.int32),
        pltpu.VMEM((CHUNK, F), jnp.float32),
        pltpu.VMEM_SHARED((N, F), jnp.float32),
        pltpu.SemaphoreType.DMA,
    ],
)
def _sc_scatter(tfeat_hbm, gidx_hbm, dst_hbm, out_hbm,
                gidx_v, dst_v, rows_v, acc_sh, sem):
    cid = lax.axis_index("c")
    sid = lax.axis_index("s")
    wid = cid * NS + sid

    zero = jnp.zeros((16,), jnp.float32)

    def _zrow(i, carry):
        for k in range(F // 16):
            rows_v[i, pl.ds(k * 16, 16)] = zero
        return carry

    lax.fori_loop(0, CHUNK, _zrow, 0)
    nmine = (ROWCHUNKS - sid + NS - 1) // NS

    def _zcopy(t, carry):
        r0 = pl.multiple_of((sid + t * NS) * CHUNK, 8)
        pltpu.sync_copy(rows_v, acc_sh.at[pl.ds(r0, CHUNK)])
        return carry

    lax.fori_loop(0, nmine, _zcopy, 0)
    plsc.subcore_barrier()

    base = wid * EDGES_PER_W

    def _chunk(j, carry):
        off = pl.multiple_of(base + j * CHUNK, 8)
        pltpu.sync_copy(gidx_hbm.at[pl.ds(off, CHUNK)], gidx_v)
        pltpu.sync_copy(dst_hbm.at[pl.ds(off, CHUNK)], dst_v)
        pltpu.async_copy(tfeat_hbm.at[gidx_v], rows_v, sem).wait()
        pltpu.sync_copy(rows_v, acc_sh.at[dst_v], add=True)
        return carry

    lax.fori_loop(0, NCHUNK, _chunk, 0)
    plsc.subcore_barrier()

    def _wcopy(t, carry):
        r0 = pl.multiple_of((sid + t * NS) * CHUNK, 8)
        pltpu.sync_copy(acc_sh.at[pl.ds(r0, CHUNK)], rows_v)
        pltpu.sync_copy(rows_v, out_hbm.at[cid, pl.ds(r0, CHUNK)])
        return carry

    lax.fori_loop(0, nmine, _wcopy, 0)


def _merge_body(p_ref, out_ref):
    out_ref[...] = p_ref[0] + p_ref[1]


_MB = 2000


@jax.jit
def _merge(partials):
    return pl.pallas_call(
        _merge_body,
        grid=(N // _MB,),
        in_specs=[pl.BlockSpec((NC, _MB, F), lambda i: (0, i, 0))],
        out_specs=pl.BlockSpec((_MB, F), lambda i: (i, 0)),
        out_shape=jax.ShapeDtypeStruct((N, F), jnp.float32),
        compiler_params=pltpu.CompilerParams(
            dimension_semantics=("parallel",)),
    )(partials)


def kernel(feat, edge_index, etypes, weight):
    tfeat = _relmm(feat, weight)
    gidx = etypes * N + edge_index[0]
    partials = _sc_scatter(tfeat.reshape(R * N, F), gidx, edge_index[1])
    return _merge(partials)
